# baseline (speedup 1.0000x reference)
import functools

import jax
import jax.numpy as jnp
from jax import lax
from jax.experimental import pallas as pl
from jax.experimental.pallas import tpu as pltpu
from jax.experimental.pallas import tpu_sc as plsc

NC = 2
NS = 16
NW = NC * NS
BE = 128
BR = 512
LANES = 16
SPLIT = (80, 80)


def _worker_range(c, s, nb0, nb1):
    base = jnp.where(c == 0, s * nb0, NS * nb0 + s * nb1)
    my_nb = jnp.where(c == 0, nb0, nb1)
    return base, my_nb


def _sc_degree(dstp2, zeros, n_pad, nb0, nb1, h):
    nb_max = max(nb0, nb1)
    mesh = plsc.VectorSubcoreMesh(core_axis_name="c", subcore_axis_name="s")

    @functools.partial(
        pl.kernel,
        out_type=jax.ShapeDtypeStruct((NC, n_pad, h), jnp.float32),
        mesh=mesh,
        scratch_types=[
            pltpu.VMEM((nb_max, BE), jnp.int32),
            pltpu.VMEM((BE, h), jnp.float32),
            pltpu.VMEM_SHARED((n_pad, h), jnp.float32),
            pltpu.SemaphoreType.DMA,
        ],
    )
    def deg_kernel(dst_hbm, z_hbm, out_hbm, idst_v, ones_v, acc_sh, sem_s):
        c = lax.axis_index("c")
        s = lax.axis_index("s")
        base, my_nb = _worker_range(c, s, nb0, nb1)
        one16 = jnp.ones((LANES,), jnp.float32)

        def fill_ones(i, carry):
            for j in range(h // LANES):
                ones_v[i, pl.ds(j * LANES, LANES)] = one16
            return carry

        lax.fori_loop(0, BE, fill_ones, 0)
        pltpu.sync_copy(dst_hbm.at[pl.ds(base, nb_max)], idst_v)

        @pl.when(s == 0)
        def _():
            pltpu.sync_copy(z_hbm, acc_sh)
        plsc.subcore_barrier()

        def scat(i):
            return pltpu.make_async_copy(
                ones_v, acc_sh.at[idst_v.at[i]], sem_s)

        scat(0).start(add=True)

        def step(i, carry):
            @pl.when(i < my_nb)
            def _():
                scat(i).start(add=True)
                scat(i - 1).wait()
            return carry

        lax.fori_loop(1, nb_max, step, 0)
        scat(my_nb - 1).wait()
        plsc.subcore_barrier()

        @pl.when(s == 0)
        def _():
            pltpu.sync_copy(acc_sh, out_hbm.at[c])

    return deg_kernel(dstp2, zeros)


def _sc_aggregate(g, srcp2, dstp2, zeros, n_pad, nb0, nb1, h):
    nb_max = max(nb0, nb1)
    mesh = plsc.VectorSubcoreMesh(core_axis_name="c", subcore_axis_name="s")

    @functools.partial(
        pl.kernel,
        out_type=jax.ShapeDtypeStruct((NC, n_pad, h), jnp.float32),
        mesh=mesh,
        scratch_types=[
            pltpu.VMEM((2, BE), jnp.int32),
            pltpu.VMEM((nb_max, BE), jnp.int32),
            pltpu.VMEM((2, BE, h), jnp.float32),
            pltpu.VMEM_SHARED((n_pad, h), jnp.float32),
            pltpu.SemaphoreType.DMA,
            pltpu.SemaphoreType.DMA,
        ],
    )
    def agg_kernel(g_hbm, src_hbm, dst_hbm, z_hbm, out_hbm,
                   isrc_v, idst_v, rows_v, acc_sh, sem_g, sem_s):
        c = lax.axis_index("c")
        s = lax.axis_index("s")
        base, my_nb = _worker_range(c, s, nb0, nb1)
        pltpu.sync_copy(dst_hbm.at[pl.ds(base, nb_max)], idst_v)

        @pl.when(s == 0)
        def _():
            pltpu.sync_copy(z_hbm, acc_sh)
        plsc.subcore_barrier()

        def gath(slot):
            return pltpu.make_async_copy(
                g_hbm.at[isrc_v.at[slot]], rows_v.at[slot], sem_g)

        def scat(i, slot):
            return pltpu.make_async_copy(
                rows_v.at[slot], acc_sh.at[idst_v.at[i]], sem_s)

        pltpu.sync_copy(src_hbm.at[pl.ds(base * BE, BE)], isrc_v.at[0])
        gath(0).start()

        def step(i, carry):
            slot = lax.rem(i, 2)
            nslot = lax.rem(i + 1, 2)

            @pl.when(i < my_nb)
            def _():
                @pl.when(i + 1 < my_nb)
                def _():
                    pltpu.sync_copy(src_hbm.at[pl.ds((base + i + 1) * BE, BE)],
                                    isrc_v.at[nslot])

                gath(slot).wait()

                @pl.when(i + 1 < my_nb)
                def _():
                    gath(nslot).start()

                d = scat(i, slot)
                d.start(add=True)
                d.wait()
            return carry

        lax.fori_loop(0, nb_max, step, 0)
        plsc.subcore_barrier()

        @pl.when(s == 0)
        def _():
            pltpu.sync_copy(acc_sh, out_hbm.at[c])

    return agg_kernel(g, srcp2, dstp2, zeros)


def _tc_first(xp, degp, W1, n, n_pad, h):
    grid = n_pad // BR

    def body(x_ref, deg_ref, w_ref, g_ref, dinv_ref):
        i = pl.program_id(0)
        rows = i * BR + lax.broadcasted_iota(jnp.int32, (BR, 1), 0)
        deg = deg_ref[0, :, 0:1] + deg_ref[1, :, 0:1] + 1.0
        dinv = jnp.where(rows < n, lax.rsqrt(deg), 0.0)
        dinv_ref[...] = dinv
        g_ref[...] = dinv * jnp.dot(x_ref[...], w_ref[...],
                                    preferred_element_type=jnp.float32)

    return pl.pallas_call(
        body,
        grid=(grid,),
        in_specs=[
            pl.BlockSpec((BR, h), lambda i: (i, 0)),
            pl.BlockSpec((NC, BR, h), lambda i: (0, i, 0)),
            pl.BlockSpec((h, h), lambda i: (0, 0)),
        ],
        out_specs=[
            pl.BlockSpec((BR, h), lambda i: (i, 0)),
            pl.BlockSpec((BR, 1), lambda i: (i, 0)),
        ],
        out_shape=[
            jax.ShapeDtypeStruct((n_pad, h), jnp.float32),
            jax.ShapeDtypeStruct((n_pad, 1), jnp.float32),
        ],
    )(xp, degp, W1)


def _tc_mid(acc, gprev, dinv, b, W, n_pad, h):
    grid = n_pad // BR

    def body(acc_ref, g_ref, dinv_ref, b_ref, w_ref, out_ref):
        dinv_blk = dinv_ref[...]
        hpre = dinv_blk * (acc_ref[0] + acc_ref[1] + g_ref[...]) + b_ref[...]
        hact = jnp.maximum(hpre, 0.0)
        out_ref[...] = dinv_blk * jnp.dot(hact, w_ref[...],
                                          preferred_element_type=jnp.float32)

    return pl.pallas_call(
        body,
        grid=(grid,),
        in_specs=[
            pl.BlockSpec((NC, BR, h), lambda i: (0, i, 0)),
            pl.BlockSpec((BR, h), lambda i: (i, 0)),
            pl.BlockSpec((BR, 1), lambda i: (i, 0)),
            pl.BlockSpec((1, h), lambda i: (0, 0)),
            pl.BlockSpec((h, h), lambda i: (0, 0)),
        ],
        out_specs=pl.BlockSpec((BR, h), lambda i: (i, 0)),
        out_shape=jax.ShapeDtypeStruct((n_pad, h), jnp.float32),
    )(acc, gprev, dinv, b, W)


def _tc_pool(acc, g3, dinv, b3, batchp, Wl, bl, n_pad, h, ngr, ncls):
    grid = n_pad // BR

    def body(acc_ref, g_ref, dinv_ref, b_ref, batch_ref, wl_ref, bl_ref,
             out_ref, sums, cnts):
        i = pl.program_id(0)
        h4 = dinv_ref[...] * (acc_ref[0] + acc_ref[1] + g_ref[...]) + b_ref[...]
        cols = lax.broadcasted_iota(jnp.int32, (BR, ngr), 1)
        onehot = (batch_ref[...] == cols).astype(jnp.float32)
        part = lax.dot_general(onehot, h4, (((0,), (0,)), ((), ())),
                               preferred_element_type=jnp.float32)
        pcnt = lax.dot_general(onehot, jnp.ones((BR, h), jnp.float32),
                               (((0,), (0,)), ((), ())),
                               preferred_element_type=jnp.float32)

        @pl.when(i == 0)
        def _():
            sums[...] = part
            cnts[...] = pcnt

        @pl.when(i > 0)
        def _():
            sums[...] = sums[...] + part
            cnts[...] = cnts[...] + pcnt

        @pl.when(i == pl.num_programs(0) - 1)
        def _():
            pooled = sums[...] / jnp.maximum(cnts[...], 1.0)
            out_ref[...] = jnp.dot(pooled, wl_ref[...],
                                   preferred_element_type=jnp.float32) + bl_ref[...]

    return pl.pallas_call(
        body,
        grid=(grid,),
        in_specs=[
            pl.BlockSpec((NC, BR, h), lambda i: (0, i, 0)),
            pl.BlockSpec((BR, h), lambda i: (i, 0)),
            pl.BlockSpec((BR, 1), lambda i: (i, 0)),
            pl.BlockSpec((1, h), lambda i: (0, 0)),
            pl.BlockSpec((BR, 1), lambda i: (i, 0)),
            pl.BlockSpec((h, ncls), lambda i: (0, 0)),
            pl.BlockSpec((1, ncls), lambda i: (0, 0)),
        ],
        out_specs=pl.BlockSpec((ngr, ncls), lambda i: (0, 0)),
        out_shape=jax.ShapeDtypeStruct((ngr, ncls), jnp.float32),
        scratch_shapes=[
            pltpu.VMEM((ngr, h), jnp.float32),
            pltpu.VMEM((ngr, h), jnp.float32),
        ],
    )(acc, g3, dinv, b3, batchp, Wl, bl)


def kernel(x, edge_index, batch, W1, b1, W2, b2, W3, b3, Wl, bl):
    n, d = x.shape
    h = W1.shape[1]
    ncls = Wl.shape[1]
    ngr = 64
    e = edge_index.shape[1]

    n_pad = ((n + BR - 1) // BR) * BR
    nb0, nb1 = SPLIT
    nbatch = NS * (nb0 + nb1)
    e_pad = nbatch * BE
    assert e_pad >= e

    src = edge_index[0]
    dst = edge_index[1]
    pad_e = e_pad - e
    srcp = jnp.concatenate([src, jnp.zeros((pad_e,), jnp.int32)])
    dstp = jnp.concatenate([dst, jnp.full((pad_e,), n, jnp.int32)]).reshape(nbatch, BE)
    xp = jnp.pad(x, ((0, n_pad - n), (0, 0)))
    batchp = jnp.pad(batch, (0, n_pad - n), constant_values=ngr).reshape(n_pad, 1)
    zeros = jnp.zeros((n_pad, h), jnp.float32)
    b1r = b1.reshape(1, h)
    b2r = b2.reshape(1, h)
    b3r = b3.reshape(1, h)
    blr = bl.reshape(1, ncls)

    degp = _sc_degree(dstp, zeros, n_pad, nb0, nb1, h)
    g1, dinv = _tc_first(xp, degp, W1, n, n_pad, h)
    acc1 = _sc_aggregate(g1, srcp, dstp, zeros, n_pad, nb0, nb1, h)
    g2 = _tc_mid(acc1, g1, dinv, b1r, W2, n_pad, h)
    acc2 = _sc_aggregate(g2, srcp, dstp, zeros, n_pad, nb0, nb1, h)
    g3 = _tc_mid(acc2, g2, dinv, b2r, W3, n_pad, h)
    acc3 = _sc_aggregate(g3, srcp, dstp, zeros, n_pad, nb0, nb1, h)
    return _tc_pool(acc3, g3, dinv, b3r, batchp, Wl, blr,
                    n_pad=n_pad, h=h, ngr=ngr, ncls=ncls)


# speedup vs baseline: 1.5062x; 1.5062x over previous
import functools

import jax
import jax.numpy as jnp
from jax import lax
from jax.experimental import pallas as pl
from jax.experimental.pallas import tpu as pltpu
from jax.experimental.pallas import tpu_sc as plsc

NC = 2
NS = 16
NW = NC * NS
BE = 128
BR = 512
LANES = 16


def _sc_degree(dstp3, zeros, n_pad, ew, h):
    nb = ew // BE
    mesh = plsc.VectorSubcoreMesh(core_axis_name="c", subcore_axis_name="s")

    @functools.partial(
        pl.kernel,
        out_type=jax.ShapeDtypeStruct((NC, n_pad, h), jnp.float32),
        mesh=mesh,
        scratch_types=[
            pltpu.VMEM((nb, BE), jnp.int32),
            pltpu.VMEM((BE, h), jnp.float32),
            pltpu.VMEM_SHARED((n_pad, h), jnp.float32),
            pltpu.SemaphoreType.DMA,
        ],
    )
    def deg_kernel(dst_hbm, z_hbm, out_hbm, idst_v, ones_v, acc_sh, sem_s):
        c = lax.axis_index("c")
        s = lax.axis_index("s")
        w = c * NS + s
        one16 = jnp.ones((LANES,), jnp.float32)

        def fill_ones(i, carry):
            for j in range(h // LANES):
                ones_v[i, pl.ds(j * LANES, LANES)] = one16
            return carry

        lax.fori_loop(0, BE, fill_ones, 0)
        pltpu.sync_copy(dst_hbm.at[w], idst_v)

        @pl.when(s == 0)
        def _():
            pltpu.sync_copy(z_hbm, acc_sh)
        plsc.subcore_barrier()

        def scat(i):
            return pltpu.make_async_copy(
                ones_v, acc_sh.at[idst_v.at[i]], sem_s)

        scat(0).start(add=True)

        def step(i, carry):
            scat(i).start(add=True)
            scat(i - 1).wait()
            return carry

        lax.fori_loop(1, nb, step, 0)
        scat(nb - 1).wait()
        plsc.subcore_barrier()

        @pl.when(s == 0)
        def _():
            pltpu.sync_copy(acc_sh, out_hbm.at[c])

    return deg_kernel(dstp3, zeros)


def _sc_aggregate(g, srcp3, dstp3, zeros, n_pad, ew, h):
    nb = ew // BE
    mesh = plsc.VectorSubcoreMesh(core_axis_name="c", subcore_axis_name="s")

    @functools.partial(
        pl.kernel,
        out_type=jax.ShapeDtypeStruct((NC, n_pad, h), jnp.float32),
        mesh=mesh,
        scratch_types=[
            pltpu.VMEM((2, BE), jnp.int32),
            pltpu.VMEM((nb, BE), jnp.int32),
            pltpu.VMEM((2, BE, h), jnp.float32),
            pltpu.VMEM_SHARED((n_pad, h), jnp.float32),
            pltpu.SemaphoreType.DMA,
            pltpu.SemaphoreType.DMA,
        ],
    )
    def agg_kernel(g_hbm, src_hbm, dst_hbm, z_hbm, out_hbm,
                   isrc_v, idst_v, rows_v, acc_sh, sem_g, sem_s):
        c = lax.axis_index("c")
        s = lax.axis_index("s")
        w = c * NS + s
        pltpu.sync_copy(dst_hbm.at[w], idst_v)

        @pl.when(s == 0)
        def _():
            pltpu.sync_copy(z_hbm, acc_sh)
        plsc.subcore_barrier()

        def gath(slot):
            return pltpu.make_async_copy(
                g_hbm.at[isrc_v.at[slot]], rows_v.at[slot], sem_g)

        def scat(i, slot):
            return pltpu.make_async_copy(
                rows_v.at[slot], acc_sh.at[idst_v.at[i]], sem_s)

        pltpu.sync_copy(src_hbm.at[w, 0], isrc_v.at[0])
        gath(0).start()

        def step(i, carry):
            slot = lax.rem(i, 2)
            nslot = lax.rem(i + 1, 2)

            @pl.when(i + 1 < nb)
            def _():
                pltpu.sync_copy(src_hbm.at[w, i + 1], isrc_v.at[nslot])

            gath(slot).wait()

            @pl.when(i + 1 < nb)
            def _():
                gath(nslot).start()

            d = scat(i, slot)
            d.start(add=True)
            d.wait()
            return carry

        lax.fori_loop(0, nb, step, 0)
        plsc.subcore_barrier()

        @pl.when(s == 0)
        def _():
            pltpu.sync_copy(acc_sh, out_hbm.at[c])

    return agg_kernel(g, srcp3, dstp3, zeros)


def _tc_first(xp, degp, W1, n, n_pad, h):
    grid = n_pad // BR

    def body(x_ref, deg_ref, w_ref, g_ref, dinv_ref):
        i = pl.program_id(0)
        rows = i * BR + lax.broadcasted_iota(jnp.int32, (BR, 1), 0)
        deg = deg_ref[0, :, 0:1] + deg_ref[1, :, 0:1] + 1.0
        dinv = jnp.where(rows < n, lax.rsqrt(deg), 0.0)
        dinv_ref[...] = dinv
        g_ref[...] = dinv * jnp.dot(x_ref[...], w_ref[...],
                                    preferred_element_type=jnp.float32)

    return pl.pallas_call(
        body,
        grid=(grid,),
        in_specs=[
            pl.BlockSpec((BR, h), lambda i: (i, 0)),
            pl.BlockSpec((NC, BR, h), lambda i: (0, i, 0)),
            pl.BlockSpec((h, h), lambda i: (0, 0)),
        ],
        out_specs=[
            pl.BlockSpec((BR, h), lambda i: (i, 0)),
            pl.BlockSpec((BR, 1), lambda i: (i, 0)),
        ],
        out_shape=[
            jax.ShapeDtypeStruct((n_pad, h), jnp.float32),
            jax.ShapeDtypeStruct((n_pad, 1), jnp.float32),
        ],
    )(xp, degp, W1)


def _tc_mid(acc, gprev, dinv, b, W, n_pad, h):
    grid = n_pad // BR

    def body(acc_ref, g_ref, dinv_ref, b_ref, w_ref, out_ref):
        dinv_blk = dinv_ref[...]
        hpre = dinv_blk * (acc_ref[0] + acc_ref[1] + g_ref[...]) + b_ref[...]
        hact = jnp.maximum(hpre, 0.0)
        out_ref[...] = dinv_blk * jnp.dot(hact, w_ref[...],
                                          preferred_element_type=jnp.float32)

    return pl.pallas_call(
        body,
        grid=(grid,),
        in_specs=[
            pl.BlockSpec((NC, BR, h), lambda i: (0, i, 0)),
            pl.BlockSpec((BR, h), lambda i: (i, 0)),
            pl.BlockSpec((BR, 1), lambda i: (i, 0)),
            pl.BlockSpec((1, h), lambda i: (0, 0)),
            pl.BlockSpec((h, h), lambda i: (0, 0)),
        ],
        out_specs=pl.BlockSpec((BR, h), lambda i: (i, 0)),
        out_shape=jax.ShapeDtypeStruct((n_pad, h), jnp.float32),
    )(acc, gprev, dinv, b, W)


def _tc_pool(acc, g3, dinv, b3, batchp, Wl, bl, n_pad, h, ngr, ncls):
    grid = n_pad // BR

    def body(acc_ref, g_ref, dinv_ref, b_ref, batch_ref, wl_ref, bl_ref,
             out_ref, sums, cnts):
        i = pl.program_id(0)
        h4 = dinv_ref[...] * (acc_ref[0] + acc_ref[1] + g_ref[...]) + b_ref[...]
        cols = lax.broadcasted_iota(jnp.int32, (BR, ngr), 1)
        onehot = (batch_ref[...] == cols).astype(jnp.float32)
        part = lax.dot_general(onehot, h4, (((0,), (0,)), ((), ())),
                               preferred_element_type=jnp.float32)
        pcnt = lax.dot_general(onehot, jnp.ones((BR, h), jnp.float32),
                               (((0,), (0,)), ((), ())),
                               preferred_element_type=jnp.float32)

        @pl.when(i == 0)
        def _():
            sums[...] = part
            cnts[...] = pcnt

        @pl.when(i > 0)
        def _():
            sums[...] = sums[...] + part
            cnts[...] = cnts[...] + pcnt

        @pl.when(i == pl.num_programs(0) - 1)
        def _():
            pooled = sums[...] / jnp.maximum(cnts[...], 1.0)
            out_ref[...] = jnp.dot(pooled, wl_ref[...],
                                   preferred_element_type=jnp.float32) + bl_ref[...]

    return pl.pallas_call(
        body,
        grid=(grid,),
        in_specs=[
            pl.BlockSpec((NC, BR, h), lambda i: (0, i, 0)),
            pl.BlockSpec((BR, h), lambda i: (i, 0)),
            pl.BlockSpec((BR, 1), lambda i: (i, 0)),
            pl.BlockSpec((1, h), lambda i: (0, 0)),
            pl.BlockSpec((BR, 1), lambda i: (i, 0)),
            pl.BlockSpec((h, ncls), lambda i: (0, 0)),
            pl.BlockSpec((1, ncls), lambda i: (0, 0)),
        ],
        out_specs=pl.BlockSpec((ngr, ncls), lambda i: (0, 0)),
        out_shape=jax.ShapeDtypeStruct((ngr, ncls), jnp.float32),
        scratch_shapes=[
            pltpu.VMEM((ngr, h), jnp.float32),
            pltpu.VMEM((ngr, h), jnp.float32),
        ],
    )(acc, g3, dinv, b3, batchp, Wl, bl)


def kernel(x, edge_index, batch, W1, b1, W2, b2, W3, b3, Wl, bl):
    n, d = x.shape
    h = W1.shape[1]
    ncls = Wl.shape[1]
    ngr = 64
    e = edge_index.shape[1]

    n_pad = ((n + BR - 1) // BR) * BR
    ew = ((e + NW * BE - 1) // (NW * BE)) * BE
    e_pad = NW * ew

    src = edge_index[0]
    dst = edge_index[1]
    pad_e = e_pad - e
    nb = ew // BE
    srcp = jnp.concatenate([src, jnp.zeros((pad_e,), jnp.int32)]).reshape(NW, nb, BE)
    dstp = jnp.concatenate([dst, jnp.full((pad_e,), n, jnp.int32)]).reshape(NW, nb, BE)
    xp = jnp.pad(x, ((0, n_pad - n), (0, 0)))
    batchp = jnp.pad(batch, (0, n_pad - n), constant_values=ngr).reshape(n_pad, 1)
    zeros = jnp.zeros((n_pad, h), jnp.float32)
    b1r = b1.reshape(1, h)
    b2r = b2.reshape(1, h)
    b3r = b3.reshape(1, h)
    blr = bl.reshape(1, ncls)

    degp = _sc_degree(dstp, zeros, n_pad, ew, h)
    g1, dinv = _tc_first(xp, degp, W1, n, n_pad, h)
    acc1 = _sc_aggregate(g1, srcp, dstp, zeros, n_pad, ew, h)
    g2 = _tc_mid(acc1, g1, dinv, b1r, W2, n_pad, h)
    acc2 = _sc_aggregate(g2, srcp, dstp, zeros, n_pad, ew, h)
    g3 = _tc_mid(acc2, g2, dinv, b2r, W3, n_pad, h)
    acc3 = _sc_aggregate(g3, srcp, dstp, zeros, n_pad, ew, h)
    return _tc_pool(acc3, g3, dinv, b3r, batchp, Wl, blr,
                    n_pad=n_pad, h=h, ngr=ngr, ncls=ncls)


# speedup vs baseline: 1.5139x; 1.0052x over previous
import functools

import jax
import jax.numpy as jnp
from jax import lax
from jax.experimental import pallas as pl
from jax.experimental.pallas import tpu as pltpu
from jax.experimental.pallas import tpu_sc as plsc

NC = 2
NS = 16
NW = NC * NS
BE = 128
BR = 512
LANES = 16


def _sc_degree(dstp3, zeros, n_pad, ew, h):
    nb = ew // BE
    mesh = plsc.VectorSubcoreMesh(core_axis_name="c", subcore_axis_name="s")

    @functools.partial(
        pl.kernel,
        out_type=jax.ShapeDtypeStruct((NC, n_pad, h), jnp.float32),
        mesh=mesh,
        scratch_types=[
            pltpu.VMEM((nb, BE), jnp.int32),
            pltpu.VMEM((BE, h), jnp.float32),
            pltpu.VMEM_SHARED((n_pad, h), jnp.float32),
            pltpu.SemaphoreType.DMA,
        ],
    )
    def deg_kernel(dst_hbm, z_hbm, out_hbm, idst_v, ones_v, acc_sh, sem_s):
        c = lax.axis_index("c")
        s = lax.axis_index("s")
        w = c * NS + s
        one16 = jnp.ones((LANES,), jnp.float32)

        def fill_ones(i, carry):
            for j in range(h // LANES):
                ones_v[i, pl.ds(j * LANES, LANES)] = one16
            return carry

        lax.fori_loop(0, BE, fill_ones, 0)
        pltpu.sync_copy(dst_hbm.at[w], idst_v)

        @pl.when(s == 0)
        def _():
            pltpu.sync_copy(z_hbm, acc_sh)
        plsc.subcore_barrier()

        def scat(i):
            return pltpu.make_async_copy(
                ones_v, acc_sh.at[idst_v.at[i]], sem_s)

        scat(0).start(add=True)

        def step(i, carry):
            scat(i).start(add=True)
            scat(i - 1).wait()
            return carry

        lax.fori_loop(1, nb, step, 0)
        scat(nb - 1).wait()
        plsc.subcore_barrier()

        @pl.when(s == 0)
        def _():
            pltpu.sync_copy(acc_sh, out_hbm.at[c])

    return deg_kernel(dstp3, zeros)


def _sc_aggregate(g, srcp3, dstp3, zeros, n_pad, ew, h):
    nb = ew // BE
    mesh = plsc.VectorSubcoreMesh(core_axis_name="c", subcore_axis_name="s")

    @functools.partial(
        pl.kernel,
        out_type=jax.ShapeDtypeStruct((NC, n_pad, h), jnp.float32),
        mesh=mesh,
        scratch_types=[
            pltpu.VMEM((2, BE), jnp.int32),
            pltpu.VMEM((nb, BE), jnp.int32),
            pltpu.VMEM((2, BE, h), jnp.float32),
            pltpu.VMEM_SHARED((n_pad, h), jnp.float32),
            pltpu.SemaphoreType.DMA,
            pltpu.SemaphoreType.DMA,
            pltpu.SemaphoreType.DMA,
        ],
    )
    def agg_kernel(g_hbm, src_hbm, dst_hbm, z_hbm, out_hbm,
                   isrc_v, idst_v, rows_v, acc_sh, sem_g, sem_s, sem_i):
        c = lax.axis_index("c")
        s = lax.axis_index("s")
        w = c * NS + s
        pltpu.sync_copy(dst_hbm.at[w], idst_v)

        @pl.when(s == 0)
        def _():
            pltpu.sync_copy(z_hbm, acc_sh)
        plsc.subcore_barrier()

        def iload(i, slot):
            return pltpu.make_async_copy(
                src_hbm.at[w, i], isrc_v.at[slot], sem_i)

        def gath(slot):
            return pltpu.make_async_copy(
                g_hbm.at[isrc_v.at[slot]], rows_v.at[slot], sem_g)

        def scat(i, slot):
            return pltpu.make_async_copy(
                rows_v.at[slot], acc_sh.at[idst_v.at[i]], sem_s)

        iload(0, 0).start()
        iload(0, 0).wait()
        gath(0).start()

        def step(i, carry):
            slot = lax.rem(i, 2)
            nslot = lax.rem(i + 1, 2)

            @pl.when(i + 1 < nb)
            def _():
                iload(i + 1, nslot).start()

            gath(slot).wait()

            @pl.when(i >= 1)
            def _():
                scat(i - 1, nslot).wait()

            @pl.when(i + 1 < nb)
            def _():
                iload(i + 1, nslot).wait()
                gath(nslot).start()

            scat(i, slot).start(add=True)
            return carry

        lax.fori_loop(0, nb, step, 0)
        scat(nb - 1, lax.rem(nb - 1, 2)).wait()
        plsc.subcore_barrier()

        @pl.when(s == 0)
        def _():
            pltpu.sync_copy(acc_sh, out_hbm.at[c])

    return agg_kernel(g, srcp3, dstp3, zeros)


def _tc_first(xp, degp, W1, n, n_pad, h):
    grid = n_pad // BR

    def body(x_ref, deg_ref, w_ref, g_ref, dinv_ref):
        i = pl.program_id(0)
        rows = i * BR + lax.broadcasted_iota(jnp.int32, (BR, 1), 0)
        deg = deg_ref[0, :, 0:1] + deg_ref[1, :, 0:1] + 1.0
        dinv = jnp.where(rows < n, lax.rsqrt(deg), 0.0)
        dinv_ref[...] = dinv
        g_ref[...] = dinv * jnp.dot(x_ref[...], w_ref[...],
                                    preferred_element_type=jnp.float32)

    return pl.pallas_call(
        body,
        grid=(grid,),
        in_specs=[
            pl.BlockSpec((BR, h), lambda i: (i, 0)),
            pl.BlockSpec((NC, BR, h), lambda i: (0, i, 0)),
            pl.BlockSpec((h, h), lambda i: (0, 0)),
        ],
        out_specs=[
            pl.BlockSpec((BR, h), lambda i: (i, 0)),
            pl.BlockSpec((BR, 1), lambda i: (i, 0)),
        ],
        out_shape=[
            jax.ShapeDtypeStruct((n_pad, h), jnp.float32),
            jax.ShapeDtypeStruct((n_pad, 1), jnp.float32),
        ],
    )(xp, degp, W1)


def _tc_mid(acc, gprev, dinv, b, W, n_pad, h):
    grid = n_pad // BR

    def body(acc_ref, g_ref, dinv_ref, b_ref, w_ref, out_ref):
        dinv_blk = dinv_ref[...]
        hpre = dinv_blk * (acc_ref[0] + acc_ref[1] + g_ref[...]) + b_ref[...]
        hact = jnp.maximum(hpre, 0.0)
        out_ref[...] = dinv_blk * jnp.dot(hact, w_ref[...],
                                          preferred_element_type=jnp.float32)

    return pl.pallas_call(
        body,
        grid=(grid,),
        in_specs=[
            pl.BlockSpec((NC, BR, h), lambda i: (0, i, 0)),
            pl.BlockSpec((BR, h), lambda i: (i, 0)),
            pl.BlockSpec((BR, 1), lambda i: (i, 0)),
            pl.BlockSpec((1, h), lambda i: (0, 0)),
            pl.BlockSpec((h, h), lambda i: (0, 0)),
        ],
        out_specs=pl.BlockSpec((BR, h), lambda i: (i, 0)),
        out_shape=jax.ShapeDtypeStruct((n_pad, h), jnp.float32),
    )(acc, gprev, dinv, b, W)


def _tc_pool(acc, g3, dinv, b3, batchp, Wl, bl, n_pad, h, ngr, ncls):
    grid = n_pad // BR

    def body(acc_ref, g_ref, dinv_ref, b_ref, batch_ref, wl_ref, bl_ref,
             out_ref, sums, cnts):
        i = pl.program_id(0)
        h4 = dinv_ref[...] * (acc_ref[0] + acc_ref[1] + g_ref[...]) + b_ref[...]
        cols = lax.broadcasted_iota(jnp.int32, (BR, ngr), 1)
        onehot = (batch_ref[...] == cols).astype(jnp.float32)
        part = lax.dot_general(onehot, h4, (((0,), (0,)), ((), ())),
                               preferred_element_type=jnp.float32)
        pcnt = lax.dot_general(onehot, jnp.ones((BR, h), jnp.float32),
                               (((0,), (0,)), ((), ())),
                               preferred_element_type=jnp.float32)

        @pl.when(i == 0)
        def _():
            sums[...] = part
            cnts[...] = pcnt

        @pl.when(i > 0)
        def _():
            sums[...] = sums[...] + part
            cnts[...] = cnts[...] + pcnt

        @pl.when(i == pl.num_programs(0) - 1)
        def _():
            pooled = sums[...] / jnp.maximum(cnts[...], 1.0)
            out_ref[...] = jnp.dot(pooled, wl_ref[...],
                                   preferred_element_type=jnp.float32) + bl_ref[...]

    return pl.pallas_call(
        body,
        grid=(grid,),
        in_specs=[
            pl.BlockSpec((NC, BR, h), lambda i: (0, i, 0)),
            pl.BlockSpec((BR, h), lambda i: (i, 0)),
            pl.BlockSpec((BR, 1), lambda i: (i, 0)),
            pl.BlockSpec((1, h), lambda i: (0, 0)),
            pl.BlockSpec((BR, 1), lambda i: (i, 0)),
            pl.BlockSpec((h, ncls), lambda i: (0, 0)),
            pl.BlockSpec((1, ncls), lambda i: (0, 0)),
        ],
        out_specs=pl.BlockSpec((ngr, ncls), lambda i: (0, 0)),
        out_shape=jax.ShapeDtypeStruct((ngr, ncls), jnp.float32),
        scratch_shapes=[
            pltpu.VMEM((ngr, h), jnp.float32),
            pltpu.VMEM((ngr, h), jnp.float32),
        ],
    )(acc, g3, dinv, b3, batchp, Wl, bl)


def kernel(x, edge_index, batch, W1, b1, W2, b2, W3, b3, Wl, bl):
    n, d = x.shape
    h = W1.shape[1]
    ncls = Wl.shape[1]
    ngr = 64
    e = edge_index.shape[1]

    n_pad = ((n + BR - 1) // BR) * BR
    ew = ((e + NW * BE - 1) // (NW * BE)) * BE
    e_pad = NW * ew

    src = edge_index[0]
    dst = edge_index[1]
    pad_e = e_pad - e
    nb = ew // BE
    srcp = jnp.concatenate([src, jnp.zeros((pad_e,), jnp.int32)]).reshape(NW, nb, BE)
    dstp = jnp.concatenate([dst, jnp.full((pad_e,), n, jnp.int32)]).reshape(NW, nb, BE)
    xp = jnp.pad(x, ((0, n_pad - n), (0, 0)))
    batchp = jnp.pad(batch, (0, n_pad - n), constant_values=ngr).reshape(n_pad, 1)
    zeros = jnp.zeros((n_pad, h), jnp.float32)
    b1r = b1.reshape(1, h)
    b2r = b2.reshape(1, h)
    b3r = b3.reshape(1, h)
    blr = bl.reshape(1, ncls)

    degp = _sc_degree(dstp, zeros, n_pad, ew, h)
    g1, dinv = _tc_first(xp, degp, W1, n, n_pad, h)
    acc1 = _sc_aggregate(g1, srcp, dstp, zeros, n_pad, ew, h)
    g2 = _tc_mid(acc1, g1, dinv, b1r, W2, n_pad, h)
    acc2 = _sc_aggregate(g2, srcp, dstp, zeros, n_pad, ew, h)
    g3 = _tc_mid(acc2, g2, dinv, b2r, W3, n_pad, h)
    acc3 = _sc_aggregate(g3, srcp, dstp, zeros, n_pad, ew, h)
    return _tc_pool(acc3, g3, dinv, b3r, batchp, Wl, blr,
                    n_pad=n_pad, h=h, ngr=ngr, ncls=ncls)
